# Initial kernel scaffold; baseline (speedup 1.0000x reference)
import jax
import jax.numpy as jnp
from jax.experimental import pallas as pl


def kernel(x, edge_index, subg_norm, W0, b0, W1, b1, Wf, bf):
    raise NotImplementedError("write your pallas kernel here")


# speedup vs baseline: 3.3089x; 3.3089x over previous
import functools

import jax
import jax.numpy as jnp
from jax import lax
from jax.experimental import pallas as pl
from jax.experimental.pallas import tpu as pltpu
from jax.experimental.pallas import tpu_sc as plsc

N = 10000
E = 320000
D = 128
NUM_CLASS = 64

NC = 2
NS = 16
NW = NC * NS
EPT = E // NW
CH = 128
G = 8
NCH = 80
NG = NCH // G
EPT_PAD = NCH * CH

AGG_ROWS = 10112
DUMMY = 10008
ZROWS = AGG_ROWS // NS
WROWS = 624


def _seg_sum_body(h_hbm, src_hbm, dst_hbm, out_hbm,
                  sidx0, didx0, sidx1, didx1, rows0, rows1,
                  semi0, semi1, sem0, sem1, ssem0, ssem1, agg_sh):
    cid = lax.axis_index("c")
    sid = lax.axis_index("s")
    wid = cid * NS + sid

    def idx_load(g, sb, db, semi):
        pltpu.async_copy(src_hbm.at[wid].at[g], sb, semi)
        pltpu.async_copy(dst_hbm.at[wid].at[g], db, semi)

    def idx_wait(g, sb, db, semi):
        pltpu.make_async_copy(src_hbm.at[wid].at[g], sb, semi).wait()
        pltpu.make_async_copy(dst_hbm.at[wid].at[g], db, semi).wait()

    zvec = jnp.zeros((16,), jnp.float32)

    def _zrow(i, _):
        for j in range(D // 16):
            rows0[i, pl.ds(j * 16, 16)] = zvec
        return 0

    lax.fori_loop(0, CH, _zrow, 0)
    zoff = sid * ZROWS
    for k in range(ZROWS // CH):
        pltpu.sync_copy(rows0.at[pl.ds(0, CH)],
                        agg_sh.at[pl.ds(zoff + k * CH, CH)])
    if ZROWS % CH:
        pltpu.sync_copy(rows0.at[pl.ds(0, ZROWS % CH)],
                        agg_sh.at[pl.ds(zoff + (ZROWS // CH) * CH, ZROWS % CH)])
    plsc.subcore_barrier()

    idx_load(0, sidx0, didx0, semi0)
    idx_load(1, sidx1, didx1, semi1)

    for p, (sb, db, semi) in enumerate(((sidx0, didx0, semi0),
                                        (sidx1, didx1, semi1))):
        def _group(k, _, p=p, sb=sb, db=db, semi=semi):
            g = 2 * k + p
            idx_wait(g, sb, db, semi)
            bufs = lambda c: ((rows0, sem0, ssem0) if c % 2 == 0
                              else (rows1, sem1, ssem1))
            pltpu.async_copy(h_hbm.at[sb.at[0]], rows0, sem0)
            for c in range(G):
                rb, sem, ssem = bufs(c)
                pltpu.make_async_copy(h_hbm.at[sb.at[c]], rb, sem).wait()
                if c >= 1:
                    pb, _, pssem = bufs(c - 1)
                    pltpu.make_async_copy(
                        pb, agg_sh.at[db.at[c - 1]], pssem).wait()
                pltpu.async_copy(rb, agg_sh.at[db.at[c]], ssem, add=True)
                if c + 1 < G:
                    nb, nsem, _ = bufs(c + 1)
                    pltpu.async_copy(h_hbm.at[sb.at[c + 1]], nb, nsem)
            lb, _, lssem = bufs(G - 1)
            pltpu.make_async_copy(lb, agg_sh.at[db.at[G - 1]], lssem).wait()

            @pl.when(k + 1 < NG // 2)
            def _():
                idx_load(g + 2, sb, db, semi)

            return 0

        lax.fori_loop(0, NG // 2, _group, 0)

    plsc.subcore_barrier()
    woff = sid * WROWS
    pltpu.sync_copy(agg_sh.at[pl.ds(woff, WROWS)],
                    out_hbm.at[cid].at[pl.ds(woff, WROWS)])

    @pl.when(sid == NS - 1)
    def _():
        pltpu.sync_copy(agg_sh.at[pl.ds(NS * WROWS, N - NS * WROWS)],
                        out_hbm.at[cid].at[pl.ds(NS * WROWS, N - NS * WROWS)])


@functools.cache
def _get_seg_sum():
    return pl.kernel(
        _seg_sum_body,
        out_type=jax.ShapeDtypeStruct((NC, N, D), jnp.float32),
        mesh=plsc.VectorSubcoreMesh(core_axis_name="c", subcore_axis_name="s",
                                    num_cores=NC),
        scratch_types=[
            pltpu.VMEM((G, CH), jnp.int32),
            pltpu.VMEM((G, CH), jnp.int32),
            pltpu.VMEM((G, CH), jnp.int32),
            pltpu.VMEM((G, CH), jnp.int32),
            pltpu.VMEM((CH, D), jnp.float32),
            pltpu.VMEM((CH, D), jnp.float32),
            pltpu.SemaphoreType.DMA,
            pltpu.SemaphoreType.DMA,
            pltpu.SemaphoreType.DMA,
            pltpu.SemaphoreType.DMA,
            pltpu.SemaphoreType.DMA,
            pltpu.SemaphoreType.DMA,
            pltpu.VMEM_SHARED((AGG_ROWS, D), jnp.float32),
        ],
    )


def _layer_body(h_ref, agg_ref, sn_ref, w_ref, b_ref, o_ref):
    h = h_ref[...]
    a = agg_ref[0] + agg_ref[1]
    hn = (a - h) * sn_ref[...]
    z = (jnp.dot(h, w_ref[0:D], preferred_element_type=jnp.float32)
         + jnp.dot(hn, w_ref[D:2 * D], preferred_element_type=jnp.float32)
         + b_ref[...])
    o_ref[...] = jnp.maximum(z, 0.0)


def _final_body(h_ref, wf_ref, bf_ref, o_ref):
    o_ref[...] = (jnp.dot(h_ref[...], wf_ref[...],
                          preferred_element_type=jnp.float32) + bf_ref[...])


_R = 400
_GRID = N // _R

_layer_call = pl.pallas_call(
    _layer_body,
    grid=(_GRID,),
    in_specs=[
        pl.BlockSpec((_R, D), lambda i: (i, 0)),
        pl.BlockSpec((NC, _R, D), lambda i: (0, i, 0)),
        pl.BlockSpec((_R, 1), lambda i: (i, 0)),
        pl.BlockSpec((2 * D, D), lambda i: (0, 0)),
        pl.BlockSpec((1, D), lambda i: (0, 0)),
    ],
    out_specs=pl.BlockSpec((_R, D), lambda i: (i, 0)),
    out_shape=jax.ShapeDtypeStruct((N, D), jnp.float32),
)

_final_call = pl.pallas_call(
    _final_body,
    grid=(_GRID,),
    in_specs=[
        pl.BlockSpec((_R, D), lambda i: (i, 0)),
        pl.BlockSpec((D, NUM_CLASS), lambda i: (0, 0)),
        pl.BlockSpec((1, NUM_CLASS), lambda i: (0, 0)),
    ],
    out_specs=pl.BlockSpec((_R, NUM_CLASS), lambda i: (i, 0)),
    out_shape=jax.ShapeDtypeStruct((N, NUM_CLASS), jnp.float32),
)


def kernel(x, edge_index, subg_norm, W0, b0, W1, b1, Wf, bf):
    src = edge_index[0].astype(jnp.int32).reshape(NW, EPT)
    dst = edge_index[1].astype(jnp.int32).reshape(NW, EPT)
    pad = EPT_PAD - EPT
    src3 = jnp.pad(src, ((0, 0), (0, pad))).reshape(NW, NG, G, CH)
    dst3 = jnp.pad(dst, ((0, 0), (0, pad)),
                   constant_values=DUMMY).reshape(NW, NG, G, CH)

    sn = subg_norm
    Ws = jnp.stack([W0, W1])
    bs = jnp.stack([b0.reshape(1, D), b1.reshape(1, D)])
    bfr = bf.reshape(1, NUM_CLASS)

    seg_sum = _get_seg_sum()

    def _step(h, wb):
        w, b = wb
        agg = seg_sum(h, src3, dst3)
        return _layer_call(h, agg, sn, w, b), None

    h2, _ = lax.scan(_step, x, (Ws, bs))
    return _final_call(h2, Wf, bfr)


# speedup vs baseline: 3.4131x; 1.0315x over previous
import functools

import jax
import jax.numpy as jnp
from jax import lax
from jax.experimental import pallas as pl
from jax.experimental.pallas import tpu as pltpu
from jax.experimental.pallas import tpu_sc as plsc

N = 10000
E = 320000
D = 128
NUM_CLASS = 64

NC = 2
NS = 16
NW = NC * NS
EPT = E // NW
CH = 128
G = 8
NCH = 80
NG = NCH // G
EPT_PAD = NCH * CH

AGG_ROWS = 10112
DUMMY = 10008
ZROWS = AGG_ROWS // NS
WROWS = 624


def _seg_sum_body(h_hbm, src_hbm, dst_hbm, out_hbm,
                  sidx0, didx0, sidx1, didx1, rows0, rows1,
                  semi0, semi1, sem0, sem1, agg_sh):
    cid = lax.axis_index("c")
    sid = lax.axis_index("s")
    wid = cid * NS + sid

    def idx_load(g, sb, db, semi):
        pltpu.async_copy(src_hbm.at[wid].at[g], sb, semi)
        pltpu.async_copy(dst_hbm.at[wid].at[g], db, semi)

    def idx_wait(g, sb, db, semi):
        pltpu.make_async_copy(src_hbm.at[wid].at[g], sb, semi).wait()
        pltpu.make_async_copy(dst_hbm.at[wid].at[g], db, semi).wait()

    zvec = jnp.zeros((16,), jnp.float32)

    def _zrow(i, _):
        for j in range(D // 16):
            rows0[i, pl.ds(j * 16, 16)] = zvec
        return 0

    lax.fori_loop(0, CH, _zrow, 0)
    zoff = sid * ZROWS
    for k in range(ZROWS // CH):
        pltpu.sync_copy(rows0.at[pl.ds(0, CH)],
                        agg_sh.at[pl.ds(zoff + k * CH, CH)])
    if ZROWS % CH:
        pltpu.sync_copy(rows0.at[pl.ds(0, ZROWS % CH)],
                        agg_sh.at[pl.ds(zoff + (ZROWS // CH) * CH, ZROWS % CH)])
    plsc.subcore_barrier()

    idx_load(0, sidx0, didx0, semi0)
    idx_load(1, sidx1, didx1, semi1)

    for p, (sb, db, semi) in enumerate(((sidx0, didx0, semi0),
                                        (sidx1, didx1, semi1))):
        def _group(k, _, p=p, sb=sb, db=db, semi=semi):
            g = 2 * k + p
            idx_wait(g, sb, db, semi)
            pltpu.async_copy(h_hbm.at[sb.at[0]], rows0, sem0)
            pltpu.async_copy(h_hbm.at[sb.at[1]], rows1, sem1)
            for c in range(G):
                rb, sem = (rows0, sem0) if c % 2 == 0 else (rows1, sem1)
                pltpu.make_async_copy(h_hbm.at[sb.at[c]], rb, sem).wait()
                pltpu.sync_copy(rb, agg_sh.at[db.at[c]], add=True)
                if c + 2 < G:
                    pltpu.async_copy(h_hbm.at[sb.at[c + 2]], rb, sem)

            @pl.when(k + 1 < NG // 2)
            def _():
                idx_load(g + 2, sb, db, semi)

            return 0

        lax.fori_loop(0, NG // 2, _group, 0)

    plsc.subcore_barrier()
    woff = sid * WROWS
    pltpu.sync_copy(agg_sh.at[pl.ds(woff, WROWS)],
                    out_hbm.at[cid].at[pl.ds(woff, WROWS)])

    @pl.when(sid == NS - 1)
    def _():
        pltpu.sync_copy(agg_sh.at[pl.ds(NS * WROWS, N - NS * WROWS)],
                        out_hbm.at[cid].at[pl.ds(NS * WROWS, N - NS * WROWS)])


@functools.cache
def _get_seg_sum():
    return pl.kernel(
        _seg_sum_body,
        out_type=jax.ShapeDtypeStruct((NC, N, D), jnp.float32),
        mesh=plsc.VectorSubcoreMesh(core_axis_name="c", subcore_axis_name="s",
                                    num_cores=NC),
        scratch_types=[
            pltpu.VMEM((G, CH), jnp.int32),
            pltpu.VMEM((G, CH), jnp.int32),
            pltpu.VMEM((G, CH), jnp.int32),
            pltpu.VMEM((G, CH), jnp.int32),
            pltpu.VMEM((CH, D), jnp.float32),
            pltpu.VMEM((CH, D), jnp.float32),
            pltpu.SemaphoreType.DMA,
            pltpu.SemaphoreType.DMA,
            pltpu.SemaphoreType.DMA,
            pltpu.SemaphoreType.DMA,
            pltpu.VMEM_SHARED((AGG_ROWS, D), jnp.float32),
        ],
    )


def _layer_body(h_ref, agg_ref, sn_ref, w_ref, b_ref, o_ref):
    h = h_ref[...]
    a = agg_ref[0] + agg_ref[1]
    hn = (a - h) * sn_ref[...]
    z = (jnp.dot(h, w_ref[0:D], preferred_element_type=jnp.float32)
         + jnp.dot(hn, w_ref[D:2 * D], preferred_element_type=jnp.float32)
         + b_ref[...])
    o_ref[...] = jnp.maximum(z, 0.0)


def _final_body(h_ref, wf_ref, bf_ref, o_ref):
    o_ref[...] = (jnp.dot(h_ref[...], wf_ref[...],
                          preferred_element_type=jnp.float32) + bf_ref[...])


_R = 400
_GRID = N // _R

_layer_call = pl.pallas_call(
    _layer_body,
    grid=(_GRID,),
    in_specs=[
        pl.BlockSpec((_R, D), lambda i: (i, 0)),
        pl.BlockSpec((NC, _R, D), lambda i: (0, i, 0)),
        pl.BlockSpec((_R, 1), lambda i: (i, 0)),
        pl.BlockSpec((2 * D, D), lambda i: (0, 0)),
        pl.BlockSpec((1, D), lambda i: (0, 0)),
    ],
    out_specs=pl.BlockSpec((_R, D), lambda i: (i, 0)),
    out_shape=jax.ShapeDtypeStruct((N, D), jnp.float32),
)

_final_call = pl.pallas_call(
    _final_body,
    grid=(_GRID,),
    in_specs=[
        pl.BlockSpec((_R, D), lambda i: (i, 0)),
        pl.BlockSpec((D, NUM_CLASS), lambda i: (0, 0)),
        pl.BlockSpec((1, NUM_CLASS), lambda i: (0, 0)),
    ],
    out_specs=pl.BlockSpec((_R, NUM_CLASS), lambda i: (i, 0)),
    out_shape=jax.ShapeDtypeStruct((N, NUM_CLASS), jnp.float32),
)


def kernel(x, edge_index, subg_norm, W0, b0, W1, b1, Wf, bf):
    src = edge_index[0].astype(jnp.int32).reshape(NW, EPT)
    dst = edge_index[1].astype(jnp.int32).reshape(NW, EPT)
    pad = EPT_PAD - EPT
    src3 = jnp.pad(src, ((0, 0), (0, pad))).reshape(NW, NG, G, CH)
    dst3 = jnp.pad(dst, ((0, 0), (0, pad)),
                   constant_values=DUMMY).reshape(NW, NG, G, CH)

    sn = subg_norm
    Ws = jnp.stack([W0, W1])
    bs = jnp.stack([b0.reshape(1, D), b1.reshape(1, D)])
    bfr = bf.reshape(1, NUM_CLASS)

    seg_sum = _get_seg_sum()

    def _step(h, wb):
        w, b = wb
        agg = seg_sum(h, src3, dst3)
        return _layer_call(h, agg, sn, w, b), None

    h2, _ = lax.scan(_step, x, (Ws, bs))
    return _final_call(h2, Wf, bfr)


# speedup vs baseline: 3.5019x; 1.0260x over previous
import functools

import jax
import jax.numpy as jnp
from jax import lax
from jax.experimental import pallas as pl
from jax.experimental.pallas import tpu as pltpu
from jax.experimental.pallas import tpu_sc as plsc

N = 10000
E = 320000
D = 128
NUM_CLASS = 64

NC = 2
NS = 16
NW = NC * NS
EPT = E // NW
CH = 128
G = 8
NCH = 80
NG = NCH // G
EPT_PAD = NCH * CH

AGG_ROWS = 10112
DUMMY = 10008
ZROWS = AGG_ROWS // NS
WROWS = 624


def _seg_sum_body(h_hbm, src_hbm, dst_hbm, out_hbm,
                  sidx0, didx0, sidx1, didx1, rows0, rows1,
                  semi0, semi1, sem0, sem1, agg_sh):
    cid = lax.axis_index("c")
    sid = lax.axis_index("s")
    wid = cid * NS + sid

    def idx_load(g, sb, db, semi):
        pltpu.async_copy(src_hbm.at[wid].at[g], sb, semi)
        pltpu.async_copy(dst_hbm.at[wid].at[g], db, semi)

    def idx_wait(g, sb, db, semi):
        pltpu.make_async_copy(src_hbm.at[wid].at[g], sb, semi).wait()
        pltpu.make_async_copy(dst_hbm.at[wid].at[g], db, semi).wait()

    zvec = jnp.zeros((16,), jnp.float32)

    def _zrow(i, _):
        for j in range(D // 16):
            rows0[i, pl.ds(j * 16, 16)] = zvec
        return 0

    lax.fori_loop(0, CH, _zrow, 0)
    zoff = sid * ZROWS
    for k in range(ZROWS // CH):
        pltpu.sync_copy(rows0.at[pl.ds(0, CH)],
                        agg_sh.at[pl.ds(zoff + k * CH, CH)])
    if ZROWS % CH:
        pltpu.sync_copy(rows0.at[pl.ds(0, ZROWS % CH)],
                        agg_sh.at[pl.ds(zoff + (ZROWS // CH) * CH, ZROWS % CH)])
    plsc.subcore_barrier()

    idx_load(0, sidx0, didx0, semi0)
    idx_load(1, sidx1, didx1, semi1)

    for p, (sb, db, semi) in enumerate(((sidx0, didx0, semi0),
                                        (sidx1, didx1, semi1))):
        def _group(k, _, p=p, sb=sb, db=db, semi=semi):
            g = 2 * k + p
            idx_wait(g, sb, db, semi)
            pltpu.async_copy(h_hbm.at[sb.at[0]], rows0, sem0)
            pltpu.async_copy(h_hbm.at[sb.at[1]], rows1, sem1)
            for c in range(G):
                rb, sem = (rows0, sem0) if c % 2 == 0 else (rows1, sem1)
                pltpu.make_async_copy(h_hbm.at[sb.at[c]], rb, sem).wait()
                pltpu.sync_copy(rb, agg_sh.at[db.at[c]], add=True)
                if c + 2 < G:
                    pltpu.async_copy(h_hbm.at[sb.at[c + 2]], rb, sem)

            @pl.when(k + 1 < NG // 2)
            def _():
                idx_load(g + 2, sb, db, semi)

            return 0

        lax.fori_loop(0, NG // 2, _group, 0)

    plsc.subcore_barrier()
    woff = sid * WROWS
    pltpu.sync_copy(agg_sh.at[pl.ds(woff, WROWS)],
                    out_hbm.at[cid].at[pl.ds(woff, WROWS)])

    @pl.when(sid == NS - 1)
    def _():
        pltpu.sync_copy(agg_sh.at[pl.ds(NS * WROWS, N - NS * WROWS)],
                        out_hbm.at[cid].at[pl.ds(NS * WROWS, N - NS * WROWS)])


@functools.cache
def _get_seg_sum():
    return pl.kernel(
        _seg_sum_body,
        out_type=jax.ShapeDtypeStruct((NC, N, D), jnp.float32),
        mesh=plsc.VectorSubcoreMesh(core_axis_name="c", subcore_axis_name="s",
                                    num_cores=NC),
        scratch_types=[
            pltpu.VMEM((G, CH), jnp.int32),
            pltpu.VMEM((G, CH), jnp.int32),
            pltpu.VMEM((G, CH), jnp.int32),
            pltpu.VMEM((G, CH), jnp.int32),
            pltpu.VMEM((CH, D), jnp.float32),
            pltpu.VMEM((CH, D), jnp.float32),
            pltpu.SemaphoreType.DMA,
            pltpu.SemaphoreType.DMA,
            pltpu.SemaphoreType.DMA,
            pltpu.SemaphoreType.DMA,
            pltpu.VMEM_SHARED((AGG_ROWS, D), jnp.float32),
        ],
    )


def _layer_body(h_ref, agg_ref, sn_ref, w_ref, b_ref, o_ref):
    h = h_ref[...]
    a = agg_ref[0] + agg_ref[1]
    hn = (a - h) * sn_ref[...]
    z = (jnp.dot(h, w_ref[0:D], preferred_element_type=jnp.float32)
         + jnp.dot(hn, w_ref[D:2 * D], preferred_element_type=jnp.float32)
         + b_ref[...])
    o_ref[...] = jnp.maximum(z, 0.0)


def _final_body(h_ref, wf_ref, bf_ref, o_ref):
    o_ref[...] = (jnp.dot(h_ref[...], wf_ref[...],
                          preferred_element_type=jnp.float32) + bf_ref[...])


_R = 1000
_GRID = N // _R

_layer_call = pl.pallas_call(
    _layer_body,
    grid=(_GRID,),
    in_specs=[
        pl.BlockSpec((_R, D), lambda i: (i, 0)),
        pl.BlockSpec((NC, _R, D), lambda i: (0, i, 0)),
        pl.BlockSpec((_R, 1), lambda i: (i, 0)),
        pl.BlockSpec((2 * D, D), lambda i: (0, 0)),
        pl.BlockSpec((1, D), lambda i: (0, 0)),
    ],
    out_specs=pl.BlockSpec((_R, D), lambda i: (i, 0)),
    out_shape=jax.ShapeDtypeStruct((N, D), jnp.float32),
)

_final_call = pl.pallas_call(
    _final_body,
    grid=(_GRID,),
    in_specs=[
        pl.BlockSpec((_R, D), lambda i: (i, 0)),
        pl.BlockSpec((D, NUM_CLASS), lambda i: (0, 0)),
        pl.BlockSpec((1, NUM_CLASS), lambda i: (0, 0)),
    ],
    out_specs=pl.BlockSpec((_R, NUM_CLASS), lambda i: (i, 0)),
    out_shape=jax.ShapeDtypeStruct((N, NUM_CLASS), jnp.float32),
)


def kernel(x, edge_index, subg_norm, W0, b0, W1, b1, Wf, bf):
    src = edge_index[0].astype(jnp.int32).reshape(NW, EPT)
    dst = edge_index[1].astype(jnp.int32).reshape(NW, EPT)
    pad = EPT_PAD - EPT
    src3 = jnp.pad(src, ((0, 0), (0, pad))).reshape(NW, NG, G, CH)
    dst3 = jnp.pad(dst, ((0, 0), (0, pad)),
                   constant_values=DUMMY).reshape(NW, NG, G, CH)

    sn = subg_norm
    Ws = jnp.stack([W0, W1])
    bs = jnp.stack([b0.reshape(1, D), b1.reshape(1, D)])
    bfr = bf.reshape(1, NUM_CLASS)

    seg_sum = _get_seg_sum()

    def _step(h, wb):
        w, b = wb
        agg = seg_sum(h, src3, dst3)
        return _layer_call(h, agg, sn, w, b), None

    h2, _ = lax.scan(_step, x, (Ws, bs))
    return _final_call(h2, Wf, bfr)


# speedup vs baseline: 3.5320x; 1.0086x over previous
import functools

import jax
import jax.numpy as jnp
from jax import lax
from jax.experimental import pallas as pl
from jax.experimental.pallas import tpu as pltpu
from jax.experimental.pallas import tpu_sc as plsc

N = 10000
E = 320000
D = 128
NUM_CLASS = 64

NC = 2
NS = 16
NW = NC * NS
EPT = E // NW
CH = 128
G = 8
NCH = 80
NG = NCH // G
EPT_PAD = NCH * CH

AGG_ROWS = 10112
DUMMY = 10008
ZROWS = AGG_ROWS // NS
WROWS = 624


def _seg_sum_body(h_hbm, src_hbm, dst_hbm, out_hbm,
                  sidx0, didx0, sidx1, didx1, rows0, rows1,
                  semi0, semi1, sem0, sem1, agg_sh):
    cid = lax.axis_index("c")
    sid = lax.axis_index("s")
    wid = cid * NS + sid

    def idx_load(g, sb, db, semi):
        pltpu.async_copy(src_hbm.at[wid].at[g], sb, semi)
        pltpu.async_copy(dst_hbm.at[wid].at[g], db, semi)

    def idx_wait(g, sb, db, semi):
        pltpu.make_async_copy(src_hbm.at[wid].at[g], sb, semi).wait()
        pltpu.make_async_copy(dst_hbm.at[wid].at[g], db, semi).wait()

    zvec = jnp.zeros((16,), jnp.float32)

    def _zrow(i, _):
        for j in range(D // 16):
            rows0[i, pl.ds(j * 16, 16)] = zvec
        return 0

    lax.fori_loop(0, CH, _zrow, 0)
    zoff = sid * ZROWS
    for k in range(ZROWS // CH):
        pltpu.sync_copy(rows0.at[pl.ds(0, CH)],
                        agg_sh.at[pl.ds(zoff + k * CH, CH)])
    if ZROWS % CH:
        pltpu.sync_copy(rows0.at[pl.ds(0, ZROWS % CH)],
                        agg_sh.at[pl.ds(zoff + (ZROWS // CH) * CH, ZROWS % CH)])
    plsc.subcore_barrier()

    idx_load(0, sidx0, didx0, semi0)
    idx_load(1, sidx1, didx1, semi1)

    for p, (sb, db, semi) in enumerate(((sidx0, didx0, semi0),
                                        (sidx1, didx1, semi1))):
        def _group(k, _, p=p, sb=sb, db=db, semi=semi):
            g = 2 * k + p
            idx_wait(g, sb, db, semi)
            pltpu.async_copy(h_hbm.at[sb.at[0]], rows0, sem0)
            pltpu.async_copy(h_hbm.at[sb.at[1]], rows1, sem1)
            for c in range(G):
                rb, sem = (rows0, sem0) if c % 2 == 0 else (rows1, sem1)
                pltpu.make_async_copy(h_hbm.at[sb.at[c]], rb, sem).wait()
                pltpu.sync_copy(rb, agg_sh.at[db.at[c]], add=True)
                if c + 2 < G:
                    pltpu.async_copy(h_hbm.at[sb.at[c + 2]], rb, sem)

            @pl.when(k + 1 < NG // 2)
            def _():
                idx_load(g + 2, sb, db, semi)

            return 0

        lax.fori_loop(0, NG // 2, _group, 0)

    plsc.subcore_barrier()
    woff = sid * WROWS
    pltpu.sync_copy(agg_sh.at[pl.ds(woff, WROWS)],
                    out_hbm.at[cid].at[pl.ds(woff, WROWS)])

    @pl.when(sid == NS - 1)
    def _():
        pltpu.sync_copy(agg_sh.at[pl.ds(NS * WROWS, N - NS * WROWS)],
                        out_hbm.at[cid].at[pl.ds(NS * WROWS, N - NS * WROWS)])


@functools.cache
def _get_seg_sum():
    return pl.kernel(
        _seg_sum_body,
        out_type=jax.ShapeDtypeStruct((NC, N, D), jnp.float32),
        mesh=plsc.VectorSubcoreMesh(core_axis_name="c", subcore_axis_name="s",
                                    num_cores=NC),
        scratch_types=[
            pltpu.VMEM((G, CH), jnp.int32),
            pltpu.VMEM((G, CH), jnp.int32),
            pltpu.VMEM((G, CH), jnp.int32),
            pltpu.VMEM((G, CH), jnp.int32),
            pltpu.VMEM((CH, D), jnp.float32),
            pltpu.VMEM((CH, D), jnp.float32),
            pltpu.SemaphoreType.DMA,
            pltpu.SemaphoreType.DMA,
            pltpu.SemaphoreType.DMA,
            pltpu.SemaphoreType.DMA,
            pltpu.VMEM_SHARED((AGG_ROWS, D), jnp.float32),
        ],
    )


def _layer_body(h_ref, agg_ref, sn_ref, w_ref, b_ref, o_ref):
    h = h_ref[...]
    a = agg_ref[0] + agg_ref[1]
    hn = (a - h) * sn_ref[...]
    z = (jnp.dot(h, w_ref[0:D], preferred_element_type=jnp.float32)
         + jnp.dot(hn, w_ref[D:2 * D], preferred_element_type=jnp.float32)
         + b_ref[...])
    o_ref[...] = jnp.maximum(z, 0.0)


def _final_body(h_ref, wf_ref, bf_ref, o_ref):
    o_ref[...] = (jnp.dot(h_ref[...], wf_ref[...],
                          preferred_element_type=jnp.float32) + bf_ref[...])


_R = 2000
_GRID = N // _R

_layer_call = pl.pallas_call(
    _layer_body,
    grid=(_GRID,),
    in_specs=[
        pl.BlockSpec((_R, D), lambda i: (i, 0)),
        pl.BlockSpec((NC, _R, D), lambda i: (0, i, 0)),
        pl.BlockSpec((_R, 1), lambda i: (i, 0)),
        pl.BlockSpec((2 * D, D), lambda i: (0, 0)),
        pl.BlockSpec((1, D), lambda i: (0, 0)),
    ],
    out_specs=pl.BlockSpec((_R, D), lambda i: (i, 0)),
    out_shape=jax.ShapeDtypeStruct((N, D), jnp.float32),
)

_final_call = pl.pallas_call(
    _final_body,
    grid=(_GRID,),
    in_specs=[
        pl.BlockSpec((_R, D), lambda i: (i, 0)),
        pl.BlockSpec((D, NUM_CLASS), lambda i: (0, 0)),
        pl.BlockSpec((1, NUM_CLASS), lambda i: (0, 0)),
    ],
    out_specs=pl.BlockSpec((_R, NUM_CLASS), lambda i: (i, 0)),
    out_shape=jax.ShapeDtypeStruct((N, NUM_CLASS), jnp.float32),
)


def kernel(x, edge_index, subg_norm, W0, b0, W1, b1, Wf, bf):
    src = edge_index[0].astype(jnp.int32).reshape(NW, EPT)
    dst = edge_index[1].astype(jnp.int32).reshape(NW, EPT)
    pad = EPT_PAD - EPT
    src3 = jnp.pad(src, ((0, 0), (0, pad))).reshape(NW, NG, G, CH)
    dst3 = jnp.pad(dst, ((0, 0), (0, pad)),
                   constant_values=DUMMY).reshape(NW, NG, G, CH)

    sn = subg_norm
    Ws = jnp.stack([W0, W1])
    bs = jnp.stack([b0.reshape(1, D), b1.reshape(1, D)])
    bfr = bf.reshape(1, NUM_CLASS)

    seg_sum = _get_seg_sum()

    def _step(h, wb):
        w, b = wb
        agg = seg_sum(h, src3, dst3)
        return _layer_call(h, agg, sn, w, b), None

    h2, _ = lax.scan(_step, x, (Ws, bs))
    return _final_call(h2, Wf, bfr)


# speedup vs baseline: 3.5364x; 1.0012x over previous
import functools

import jax
import jax.numpy as jnp
from jax import lax
from jax.experimental import pallas as pl
from jax.experimental.pallas import tpu as pltpu
from jax.experimental.pallas import tpu_sc as plsc

N = 10000
E = 320000
D = 128
NUM_CLASS = 64

NC = 2
NS = 16
NW = NC * NS
EPT = E // NW
CH = 128
G = 8
NCH = 80
NG = NCH // G
EPT_PAD = NCH * CH

AGG_ROWS = 10112
DUMMY = 10008
ZROWS = AGG_ROWS // NS
WROWS = 624


def _seg_sum_body(h_hbm, src_hbm, dst_hbm, out_hbm,
                  sidx0, didx0, sidx1, didx1, rows0, rows1,
                  semi0, semi1, sem0, sem1, agg_sh):
    cid = lax.axis_index("c")
    sid = lax.axis_index("s")
    wid = cid * NS + sid

    def idx_load(g, sb, db, semi):
        pltpu.async_copy(src_hbm.at[wid].at[g], sb, semi)
        pltpu.async_copy(dst_hbm.at[wid].at[g], db, semi)

    def idx_wait(g, sb, db, semi):
        pltpu.make_async_copy(src_hbm.at[wid].at[g], sb, semi).wait()
        pltpu.make_async_copy(dst_hbm.at[wid].at[g], db, semi).wait()

    zvec = jnp.zeros((16,), jnp.float32)

    def _zrow(i, _):
        for j in range(D // 16):
            rows0[i, pl.ds(j * 16, 16)] = zvec
        return 0

    lax.fori_loop(0, CH, _zrow, 0)
    zoff = sid * ZROWS
    for k in range(ZROWS // CH):
        pltpu.sync_copy(rows0.at[pl.ds(0, CH)],
                        agg_sh.at[pl.ds(zoff + k * CH, CH)])
    if ZROWS % CH:
        pltpu.sync_copy(rows0.at[pl.ds(0, ZROWS % CH)],
                        agg_sh.at[pl.ds(zoff + (ZROWS // CH) * CH, ZROWS % CH)])
    plsc.subcore_barrier()

    idx_load(0, sidx0, didx0, semi0)
    idx_load(1, sidx1, didx1, semi1)

    for p, (sb, db, semi) in enumerate(((sidx0, didx0, semi0),
                                        (sidx1, didx1, semi1))):
        def _group(k, _, p=p, sb=sb, db=db, semi=semi):
            g = 2 * k + p
            idx_wait(g, sb, db, semi)
            pltpu.async_copy(h_hbm.at[sb.at[0]], rows0, sem0)
            pltpu.async_copy(h_hbm.at[sb.at[1]], rows1, sem1)
            for c in range(G):
                rb, sem = (rows0, sem0) if c % 2 == 0 else (rows1, sem1)
                pltpu.make_async_copy(h_hbm.at[sb.at[c]], rb, sem).wait()
                pltpu.sync_copy(rb, agg_sh.at[db.at[c]], add=True)
                if c + 2 < G:
                    pltpu.async_copy(h_hbm.at[sb.at[c + 2]], rb, sem)

            @pl.when(k + 1 < NG // 2)
            def _():
                idx_load(g + 2, sb, db, semi)

            return 0

        lax.fori_loop(0, NG // 2, _group, 0)

    plsc.subcore_barrier()
    woff = sid * WROWS
    pltpu.sync_copy(agg_sh.at[pl.ds(woff, WROWS)],
                    out_hbm.at[cid].at[pl.ds(woff, WROWS)])

    @pl.when(sid == NS - 1)
    def _():
        pltpu.sync_copy(agg_sh.at[pl.ds(NS * WROWS, N - NS * WROWS)],
                        out_hbm.at[cid].at[pl.ds(NS * WROWS, N - NS * WROWS)])


@functools.cache
def _get_seg_sum():
    return pl.kernel(
        _seg_sum_body,
        out_type=jax.ShapeDtypeStruct((NC, N, D), jnp.float32),
        mesh=plsc.VectorSubcoreMesh(core_axis_name="c", subcore_axis_name="s",
                                    num_cores=NC),
        scratch_types=[
            pltpu.VMEM((G, CH), jnp.int32),
            pltpu.VMEM((G, CH), jnp.int32),
            pltpu.VMEM((G, CH), jnp.int32),
            pltpu.VMEM((G, CH), jnp.int32),
            pltpu.VMEM((CH, D), jnp.float32),
            pltpu.VMEM((CH, D), jnp.float32),
            pltpu.SemaphoreType.DMA,
            pltpu.SemaphoreType.DMA,
            pltpu.SemaphoreType.DMA,
            pltpu.SemaphoreType.DMA,
            pltpu.VMEM_SHARED((AGG_ROWS, D), jnp.float32),
        ],
    )


def _layer_body(h_ref, agg_ref, sn_ref, w_ref, b_ref, o_ref):
    h = h_ref[...]
    a = agg_ref[0] + agg_ref[1]
    hn = (a - h) * sn_ref[...]
    z = (jnp.dot(h, w_ref[0:D], preferred_element_type=jnp.float32)
         + jnp.dot(hn, w_ref[D:2 * D], preferred_element_type=jnp.float32)
         + b_ref[...])
    o_ref[...] = jnp.maximum(z, 0.0)


def _final_body(h_ref, wf_ref, bf_ref, o_ref):
    o_ref[...] = (jnp.dot(h_ref[...], wf_ref[...],
                          preferred_element_type=jnp.float32) + bf_ref[...])


_R = 10000
_GRID = N // _R

_layer_call = pl.pallas_call(
    _layer_body,
    grid=(_GRID,),
    in_specs=[
        pl.BlockSpec((_R, D), lambda i: (i, 0)),
        pl.BlockSpec((NC, _R, D), lambda i: (0, i, 0)),
        pl.BlockSpec((_R, 1), lambda i: (i, 0)),
        pl.BlockSpec((2 * D, D), lambda i: (0, 0)),
        pl.BlockSpec((1, D), lambda i: (0, 0)),
    ],
    out_specs=pl.BlockSpec((_R, D), lambda i: (i, 0)),
    out_shape=jax.ShapeDtypeStruct((N, D), jnp.float32),
)

_final_call = pl.pallas_call(
    _final_body,
    grid=(_GRID,),
    in_specs=[
        pl.BlockSpec((_R, D), lambda i: (i, 0)),
        pl.BlockSpec((D, NUM_CLASS), lambda i: (0, 0)),
        pl.BlockSpec((1, NUM_CLASS), lambda i: (0, 0)),
    ],
    out_specs=pl.BlockSpec((_R, NUM_CLASS), lambda i: (i, 0)),
    out_shape=jax.ShapeDtypeStruct((N, NUM_CLASS), jnp.float32),
)


def kernel(x, edge_index, subg_norm, W0, b0, W1, b1, Wf, bf):
    src = edge_index[0].astype(jnp.int32).reshape(NW, EPT)
    dst = edge_index[1].astype(jnp.int32).reshape(NW, EPT)
    pad = EPT_PAD - EPT
    src3 = jnp.pad(src, ((0, 0), (0, pad))).reshape(NW, NG, G, CH)
    dst3 = jnp.pad(dst, ((0, 0), (0, pad)),
                   constant_values=DUMMY).reshape(NW, NG, G, CH)

    sn = subg_norm
    Ws = jnp.stack([W0, W1])
    bs = jnp.stack([b0.reshape(1, D), b1.reshape(1, D)])
    bfr = bf.reshape(1, NUM_CLASS)

    seg_sum = _get_seg_sum()

    def _step(h, wb):
        w, b = wb
        agg = seg_sum(h, src3, dst3)
        return _layer_call(h, agg, sn, w, b), None

    h2, _ = lax.scan(_step, x, (Ws, bs))
    return _final_call(h2, Wf, bfr)
